# baseline (speedup 1.0000x reference)
import functools

import jax
import jax.numpy as jnp
import numpy as np
from jax.experimental import pallas as pl
from jax.experimental.pallas import tpu as pltpu

_L = 5
_K = 5
_W = 30.0
_MULT = np.array([(1000003 ** k) % (2 ** 31 - 1) for k in range(_K)],
                 dtype=np.int32)

_TN = 2000


def _bucket_cols(X, A, b):
    h = jnp.floor((jax.lax.dot_general(
        X, A, (((1,), (0,)), ((), ())),
        preferred_element_type=jnp.float32) + b) / _W).astype(jnp.int32)
    cols = []
    for l in range(_L):
        acc = h[:, _K * l:_K * l + 1] * _MULT[0]
        for k in range(1, _K):
            acc = acc + h[:, _K * l + k:_K * l + k + 1] * _MULT[k]
        cols.append(acc)
    return cols


def _fused_body(u_ref, ref_ref, a_ref, b_ref, out_ref, num_ref, den_ref):
    i = pl.program_id(0)
    nt = pl.num_programs(0)

    U = u_ref[...]
    R = ref_ref[...]
    A = a_ref[...]
    b = b_ref[...]

    qb = _bucket_cols(U, A, b)
    rb = _bucket_cols(R, A, b)

    match = qb[0] == rb[0].reshape(1, _TN)
    for l in range(1, _L):
        match = match | (qb[l] == rb[l].reshape(1, _TN))

    un2 = jnp.sum(U * U, axis=1, keepdims=True)
    rn2 = jnp.sum(R * R, axis=1, keepdims=True)
    S = jax.lax.dot_general(U, R, (((1,), (1,)), ((), ())),
                            preferred_element_type=jnp.float32)
    d2 = jnp.maximum(un2 + rn2.reshape(1, _TN) - 2.0 * S, 0.0)
    w = jnp.where(match, jnp.exp(d2 * (-1.0 / (2.0 * _W))), 0.0)

    pnum = jax.lax.dot_general(w, R, (((1,), (0,)), ((), ())),
                               preferred_element_type=jnp.float32)
    pden = jnp.sum(w, axis=1, keepdims=True)

    @pl.when(i == 0)
    def _init():
        num_ref[...] = pnum
        den_ref[...] = pden

    @pl.when(i > 0)
    def _acc():
        num_ref[...] += pnum
        den_ref[...] += pden

    @pl.when(i == nt - 1)
    def _final():
        out_ref[...] = num_ref[...] / (den_ref[...] + 1e-6) - U


@jax.jit
def kernel(U, ref, A, b):
    Q, D = U.shape
    N = ref.shape[0]
    assert N % _TN == 0
    grid = (N // _TN,)
    out = pl.pallas_call(
        _fused_body,
        grid=grid,
        in_specs=[
            pl.BlockSpec((Q, D), lambda i: (0, 0)),
            pl.BlockSpec((_TN, D), lambda i: (i, 0)),
            pl.BlockSpec((D, _L * _K), lambda i: (0, 0)),
            pl.BlockSpec((1, _L * _K), lambda i: (0, 0)),
        ],
        out_shape=jax.ShapeDtypeStruct((Q, D), jnp.float32),
        scratch_shapes=[
            pltpu.VMEM((Q, D), jnp.float32),
            pltpu.VMEM((Q, 1), jnp.float32),
        ],
    )(U, ref, A, b.reshape(1, -1))
    return out


# speedup vs baseline: 1.4656x; 1.4656x over previous
import jax
import jax.numpy as jnp
import numpy as np
from jax.experimental import pallas as pl
from jax.experimental.pallas import tpu as pltpu

_L = 5
_K = 5
_W = 30.0
_MULT = np.array([(1000003 ** k) % (2 ** 31 - 1) for k in range(_K)],
                 dtype=np.int32)
_MULT_COL = np.tile(_MULT, _L).reshape(_L * _K, 1)

_TN = 2000


def _fused_body(u_ref, ref_ref, a_ref, b_ref, mult_ref, out_ref,
                num_ref, den_ref, qb_ref, fq_ref):
    i = pl.program_id(0)
    nt = pl.num_programs(0)

    U = u_ref[...]
    R = ref_ref[...]
    A = a_ref[...]
    bcol = b_ref[...]

    @pl.when(i == 0)
    def _prep():
        hq = jnp.floor((jax.lax.dot_general(
            U, A, (((1,), (0,)), ((), ())),
            preferred_element_type=jnp.float32) + bcol.reshape(1, -1))
            / _W).astype(jnp.int32)
        for l in range(_L):
            acc = hq[:, _K * l:_K * l + 1] * _MULT[0]
            for k in range(1, _K):
                acc = acc + hq[:, _K * l + k:_K * l + k + 1] * _MULT[k]
            qb_ref[:, l:l + 1] = acc
        un2 = jnp.sum(U * U, axis=1, keepdims=True)
        fq_ref[...] = jnp.exp(un2 * (-1.0 / (2.0 * _W)))

    hr = jnp.floor((jax.lax.dot_general(
        A, R, (((0,), (1,)), ((), ())),
        preferred_element_type=jnp.float32) + bcol) / _W).astype(jnp.int32)
    hm = hr * mult_ref[...]
    rb = []
    for l in range(_L):
        acc = hm[_K * l:_K * l + 1, :]
        for k in range(1, _K):
            acc = acc + hm[_K * l + k:_K * l + k + 1, :]
        rb.append(acc)

    match = qb_ref[:, 0:1] == rb[0]
    for l in range(1, _L):
        match = match | (qb_ref[:, l:l + 1] == rb[l])

    gr = jnp.exp(jax.lax.dot_general(
        jnp.ones((1, R.shape[1]), jnp.float32), R * R,
        (((1,), (1,)), ((), ())),
        preferred_element_type=jnp.float32) * (-1.0 / (2.0 * _W)))

    S = jax.lax.dot_general(U, R, (((1,), (1,)), ((), ())),
                            preferred_element_type=jnp.float32)
    w = jnp.where(match, jnp.exp(S * (1.0 / _W)) * (fq_ref[...] * gr), 0.0)

    pnum = jax.lax.dot_general(w, R, (((1,), (0,)), ((), ())),
                               preferred_element_type=jnp.float32)
    pden = jax.lax.dot_general(
        w, jnp.ones((_TN, 1), jnp.float32), (((1,), (0,)), ((), ())),
        preferred_element_type=jnp.float32)

    @pl.when(i == 0)
    def _init():
        num_ref[...] = pnum
        den_ref[...] = pden

    @pl.when(i > 0)
    def _acc():
        num_ref[...] += pnum
        den_ref[...] += pden

    @pl.when(i == nt - 1)
    def _final():
        out_ref[...] = num_ref[...] / (den_ref[...] + 1e-6) - U


@jax.jit
def kernel(U, ref, A, b):
    Q, D = U.shape
    N = ref.shape[0]
    assert N % _TN == 0
    grid = (N // _TN,)
    out = pl.pallas_call(
        _fused_body,
        grid=grid,
        in_specs=[
            pl.BlockSpec((Q, D), lambda i: (0, 0)),
            pl.BlockSpec((_TN, D), lambda i: (i, 0)),
            pl.BlockSpec((D, _L * _K), lambda i: (0, 0)),
            pl.BlockSpec((_L * _K, 1), lambda i: (0, 0)),
            pl.BlockSpec((_L * _K, 1), lambda i: (0, 0)),
        ],
        out_shape=jax.ShapeDtypeStruct((Q, D), jnp.float32),
        scratch_shapes=[
            pltpu.VMEM((Q, D), jnp.float32),
            pltpu.VMEM((Q, 1), jnp.float32),
            pltpu.VMEM((Q, _L), jnp.int32),
            pltpu.VMEM((Q, 1), jnp.float32),
        ],
    )(U, ref, A, b.reshape(-1, 1), jnp.asarray(_MULT_COL))
    return out


# speedup vs baseline: 1.4684x; 1.0019x over previous
import jax
import jax.numpy as jnp
import numpy as np
from jax.experimental import pallas as pl
from jax.experimental.pallas import tpu as pltpu

_L = 5
_K = 5
_W = 30.0
_MULT = np.array([(1000003 ** k) % (2 ** 31 - 1) for k in range(_K)],
                 dtype=np.int32)
_MULT_COL = np.tile(_MULT, _L).reshape(_L * _K, 1)
_C = float(np.log2(np.e) / _W)

_TN = 2000


def _fused_body(u_ref, uc_ref, ucol_ref, ref_ref, a_ref, b_ref, mult_ref,
                out_ref, num_ref, den_ref, qb_ref):
    i = pl.program_id(0)
    nt = pl.num_programs(0)

    U = u_ref[...]
    R = ref_ref[...]
    A = a_ref[...]
    bcol = b_ref[...]

    @pl.when(i == 0)
    def _prep():
        hq = jnp.floor((jax.lax.dot_general(
            U, A, (((1,), (0,)), ((), ())),
            preferred_element_type=jnp.float32) + bcol.reshape(1, -1))
            / _W).astype(jnp.int32)
        for l in range(_L):
            acc = hq[:, _K * l:_K * l + 1] * _MULT[0]
            for k in range(1, _K):
                acc = acc + hq[:, _K * l + k:_K * l + k + 1] * _MULT[k]
            qb_ref[:, l:l + 1] = acc

    hr = jnp.floor((jax.lax.dot_general(
        A, R, (((0,), (1,)), ((), ())),
        preferred_element_type=jnp.float32) + bcol) / _W).astype(jnp.int32)
    hm = hr * mult_ref[...]
    rb = []
    for l in range(_L):
        acc = hm[_K * l:_K * l + 1, :]
        for k in range(1, _K):
            acc = acc + hm[_K * l + k:_K * l + k + 1, :]
        rb.append(acc)

    match = qb_ref[:, 0:1] == rb[0]
    for l in range(1, _L):
        match = match | (qb_ref[:, l:l + 1] == rb[l])

    rrow = jax.lax.dot_general(
        jnp.full((1, R.shape[1]), -0.5 * _C, jnp.float32), R * R,
        (((1,), (1,)), ((), ())),
        preferred_element_type=jnp.float32)

    S = jax.lax.dot_general(uc_ref[...], R, (((1,), (1,)), ((), ())),
                            preferred_element_type=jnp.float32)
    w = jnp.where(match, jnp.exp2(S + ucol_ref[...] + rrow), 0.0)

    pnum = jax.lax.dot_general(w, R, (((1,), (0,)), ((), ())),
                               preferred_element_type=jnp.float32)
    pden = jax.lax.dot_general(
        w, jnp.ones((_TN, 1), jnp.float32), (((1,), (0,)), ((), ())),
        preferred_element_type=jnp.float32)

    @pl.when(i == 0)
    def _init():
        num_ref[...] = pnum
        den_ref[...] = pden

    @pl.when(i > 0)
    def _acc():
        num_ref[...] += pnum
        den_ref[...] += pden

    @pl.when(i == nt - 1)
    def _final():
        out_ref[...] = num_ref[...] / (den_ref[...] + 1e-6) - U


@jax.jit
def kernel(U, ref, A, b):
    Q, D = U.shape
    N = ref.shape[0]
    assert N % _TN == 0
    grid = (N // _TN,)
    Uc = U * jnp.float32(_C)
    ucol = jnp.sum(U * U, axis=1, keepdims=True) * jnp.float32(-0.5 * _C)
    out = pl.pallas_call(
        _fused_body,
        grid=grid,
        in_specs=[
            pl.BlockSpec((Q, D), lambda i: (0, 0)),
            pl.BlockSpec((Q, D), lambda i: (0, 0)),
            pl.BlockSpec((Q, 1), lambda i: (0, 0)),
            pl.BlockSpec((_TN, D), lambda i: (i, 0)),
            pl.BlockSpec((D, _L * _K), lambda i: (0, 0)),
            pl.BlockSpec((_L * _K, 1), lambda i: (0, 0)),
            pl.BlockSpec((_L * _K, 1), lambda i: (0, 0)),
        ],
        out_shape=jax.ShapeDtypeStruct((Q, D), jnp.float32),
        scratch_shapes=[
            pltpu.VMEM((Q, D), jnp.float32),
            pltpu.VMEM((Q, 1), jnp.float32),
            pltpu.VMEM((Q, _L), jnp.int32),
        ],
    )(U, Uc, ucol, ref, A, b.reshape(-1, 1), jnp.asarray(_MULT_COL))
    return out


# speedup vs baseline: 1.5184x; 1.0341x over previous
import jax
import jax.numpy as jnp
import numpy as np
from jax.experimental import pallas as pl
from jax.experimental.pallas import tpu as pltpu

_L = 5
_K = 5
_W = 30.0
_MULT = np.array([(1000003 ** k) % (2 ** 31 - 1) for k in range(_K)],
                 dtype=np.int32)
_MULT_COL = np.tile(_MULT, _L).reshape(_L * _K, 1)
_C = float(np.log2(np.e) / _W)

_TN = 4000


def _prep_body(u_ref, a_ref, b_ref, qb_ref):
    U = u_ref[...]
    hq = jnp.floor((jax.lax.dot_general(
        U, a_ref[...], (((1,), (0,)), ((), ())),
        preferred_element_type=jnp.float32) + b_ref[...].reshape(1, -1))
        / _W).astype(jnp.int32)
    for l in range(_L):
        acc = hq[:, _K * l:_K * l + 1] * _MULT[0]
        for k in range(1, _K):
            acc = acc + hq[:, _K * l + k:_K * l + k + 1] * _MULT[k]
        qb_ref[:, l:l + 1] = acc
    qb_ref[:, _L:] = jnp.zeros_like(qb_ref[:, _L:])


def _fused_body(u_ref, uc_ref, ucol_ref, qb_ref, ref_ref, a_ref, b_ref,
                mult_ref, out_ref, num_ref, den_ref):
    i = pl.program_id(0)
    nt = pl.num_programs(0)

    R = ref_ref[...]
    A = a_ref[...]
    bcol = b_ref[...]

    hr = jnp.floor((jax.lax.dot_general(
        A, R, (((0,), (1,)), ((), ())),
        preferred_element_type=jnp.float32) + bcol) / _W).astype(jnp.int32)
    hm = hr * mult_ref[...]
    rb = []
    for l in range(_L):
        acc = hm[_K * l:_K * l + 1, :]
        for k in range(1, _K):
            acc = acc + hm[_K * l + k:_K * l + k + 1, :]
        rb.append(acc)

    match = qb_ref[:, 0:1] == rb[0]
    for l in range(1, _L):
        match = match | (qb_ref[:, l:l + 1] == rb[l])

    rrow = jax.lax.dot_general(
        jnp.full((1, R.shape[1]), -0.5 * _C, jnp.float32), R * R,
        (((1,), (1,)), ((), ())),
        preferred_element_type=jnp.float32)

    S = jax.lax.dot_general(uc_ref[...], R, (((1,), (1,)), ((), ())),
                            preferred_element_type=jnp.float32)
    w = jnp.where(match, jnp.exp2(S + ucol_ref[...] + rrow), 0.0)

    pnum = jax.lax.dot_general(w, R, (((1,), (0,)), ((), ())),
                               preferred_element_type=jnp.float32)
    pden = jax.lax.dot_general(
        w, jnp.ones((_TN, 1), jnp.float32), (((1,), (0,)), ((), ())),
        preferred_element_type=jnp.float32)

    @pl.when(i == 0)
    def _init():
        num_ref[...] = pnum
        den_ref[...] = pden

    @pl.when(i > 0)
    def _acc():
        num_ref[...] += pnum
        den_ref[...] += pden

    @pl.when(i == nt - 1)
    def _final():
        out_ref[...] = num_ref[...] / (den_ref[...] + 1e-6) - u_ref[...]


@jax.jit
def kernel(U, ref, A, b):
    Q, D = U.shape
    N = ref.shape[0]
    assert N % _TN == 0
    grid = (N // _TN,)
    Uc = U * jnp.float32(_C)
    ucol = jnp.sum(U * U, axis=1, keepdims=True) * jnp.float32(-0.5 * _C)
    bcol = b.reshape(-1, 1)
    mult = jnp.asarray(_MULT_COL)

    qb = pl.pallas_call(
        _prep_body,
        in_specs=[
            pl.BlockSpec((Q, D), lambda: (0, 0)),
            pl.BlockSpec((D, _L * _K), lambda: (0, 0)),
            pl.BlockSpec((_L * _K, 1), lambda: (0, 0)),
        ],
        out_shape=jax.ShapeDtypeStruct((Q, 8), jnp.int32),
    )(U, A, bcol)

    out = pl.pallas_call(
        _fused_body,
        grid=grid,
        in_specs=[
            pl.BlockSpec((Q, D), lambda i: (0, 0)),
            pl.BlockSpec((Q, D), lambda i: (0, 0)),
            pl.BlockSpec((Q, 1), lambda i: (0, 0)),
            pl.BlockSpec((Q, 8), lambda i: (0, 0)),
            pl.BlockSpec((_TN, D), lambda i: (i, 0)),
            pl.BlockSpec((D, _L * _K), lambda i: (0, 0)),
            pl.BlockSpec((_L * _K, 1), lambda i: (0, 0)),
            pl.BlockSpec((_L * _K, 1), lambda i: (0, 0)),
        ],
        out_shape=jax.ShapeDtypeStruct((Q, D), jnp.float32),
        scratch_shapes=[
            pltpu.VMEM((Q, D), jnp.float32),
            pltpu.VMEM((Q, 1), jnp.float32),
        ],
    )(U, Uc, ucol, qb, ref, A, bcol, mult)
    return out


# speedup vs baseline: 1.6368x; 1.0780x over previous
import jax
import jax.numpy as jnp
import numpy as np
from jax.experimental import pallas as pl
from jax.experimental.pallas import tpu as pltpu

_L = 5
_K = 5
_W = 30.0
_MULT = np.array([(1000003 ** k) % (2 ** 31 - 1) for k in range(_K)],
                 dtype=np.int32)
_MULT_COL = np.tile(_MULT, _L).reshape(_L * _K, 1)
_C = float(np.log2(np.e) / _W)

_TN = 4000


def _prep_body(u_ref, a_ref, b_ref, qb_ref):
    U = u_ref[...]
    hq = jnp.floor((jax.lax.dot_general(
        U, a_ref[...], (((1,), (0,)), ((), ())),
        preferred_element_type=jnp.float32) + b_ref[...].reshape(1, -1))
        / _W).astype(jnp.int32)
    for l in range(_L):
        acc = hq[:, _K * l:_K * l + 1] * _MULT[0]
        for k in range(1, _K):
            acc = acc + hq[:, _K * l + k:_K * l + k + 1] * _MULT[k]
        qb_ref[:, l:l + 1] = acc
    qb_ref[:, _L:] = jnp.zeros_like(qb_ref[:, _L:])


def _fused_body(u_ref, uc_ref, qb_ref, ref_ref, a_ref, b_ref,
                mult_ref, out_ref, num_ref, den_ref):
    i = pl.program_id(0)
    nt = pl.num_programs(0)

    R = ref_ref[...]
    A = a_ref[...]
    bcol = b_ref[...]

    hr = jnp.floor((jax.lax.dot_general(
        A, R, (((0,), (1,)), ((), ())),
        preferred_element_type=jnp.float32) + bcol) / _W).astype(jnp.int32)
    hm = hr * mult_ref[...]
    rb = []
    for l in range(_L):
        acc = hm[_K * l:_K * l + 1, :]
        for k in range(1, _K):
            acc = acc + hm[_K * l + k:_K * l + k + 1, :]
        rb.append(acc)

    match = qb_ref[:, 0:1] == rb[0]
    for l in range(1, _L):
        match = match | (qb_ref[:, l:l + 1] == rb[l])

    rrow = jax.lax.dot_general(
        jnp.full((1, R.shape[1]), -0.5 * _C, jnp.float32), R * R,
        (((1,), (1,)), ((), ())),
        preferred_element_type=jnp.float32)

    Rb = R.astype(jnp.bfloat16)
    S = jax.lax.dot_general(uc_ref[...], Rb, (((1,), (1,)), ((), ())),
                            preferred_element_type=jnp.float32)
    w = jnp.where(match, jnp.exp2(S + rrow), 0.0).astype(jnp.bfloat16)

    pnum = jax.lax.dot_general(w, Rb, (((1,), (0,)), ((), ())),
                               preferred_element_type=jnp.float32)
    pden = jax.lax.dot_general(
        w, jnp.ones((_TN, 1), jnp.bfloat16), (((1,), (0,)), ((), ())),
        preferred_element_type=jnp.float32)

    @pl.when(i == 0)
    def _init():
        num_ref[...] = pnum
        den_ref[...] = pden

    @pl.when(i > 0)
    def _acc():
        num_ref[...] += pnum
        den_ref[...] += pden

    @pl.when(i == nt - 1)
    def _final():
        U = u_ref[...]
        un2 = jnp.sum(U * U, axis=1, keepdims=True)
        eps = jnp.exp2(un2 * (0.5 * _C)) * 1e-6
        out_ref[...] = num_ref[...] / (den_ref[...] + eps) - U


@jax.jit
def kernel(U, ref, A, b):
    Q, D = U.shape
    N = ref.shape[0]
    assert N % _TN == 0
    grid = (N // _TN,)
    Uc = (U * jnp.float32(_C)).astype(jnp.bfloat16)
    bcol = b.reshape(-1, 1)
    mult = jnp.asarray(_MULT_COL)

    qb = pl.pallas_call(
        _prep_body,
        in_specs=[
            pl.BlockSpec((Q, D), lambda: (0, 0)),
            pl.BlockSpec((D, _L * _K), lambda: (0, 0)),
            pl.BlockSpec((_L * _K, 1), lambda: (0, 0)),
        ],
        out_shape=jax.ShapeDtypeStruct((Q, 8), jnp.int32),
    )(U, A, bcol)

    out = pl.pallas_call(
        _fused_body,
        grid=grid,
        in_specs=[
            pl.BlockSpec((Q, D), lambda i: (0, 0)),
            pl.BlockSpec((Q, D), lambda i: (0, 0)),
            pl.BlockSpec((Q, 8), lambda i: (0, 0)),
            pl.BlockSpec((_TN, D), lambda i: (i, 0)),
            pl.BlockSpec((D, _L * _K), lambda i: (0, 0)),
            pl.BlockSpec((_L * _K, 1), lambda i: (0, 0)),
            pl.BlockSpec((_L * _K, 1), lambda i: (0, 0)),
        ],
        out_shape=jax.ShapeDtypeStruct((Q, D), jnp.float32),
        scratch_shapes=[
            pltpu.VMEM((Q, D), jnp.float32),
            pltpu.VMEM((Q, 1), jnp.float32),
        ],
    )(U, Uc, qb, ref, A, bcol, mult)
    return out
